# baseline (speedup 1.0000x reference)
import functools

import jax
import jax.numpy as jnp
from jax import lax
from jax.experimental import pallas as pl
from jax.experimental.pallas import tpu as pltpu
from jax.experimental.pallas import tpu_sc as plsc

N = 10000
FIN = 128
H1, C1 = 8, 16
D1 = 144
F2 = 64
D2 = 80
E_RAW = 320000
E_TOT = E_RAW + N
NC, NS, NW = 2, 16, 32
EPW = 10496
CHUNK1, CPW1 = 64, 164
S1 = 4
NSUP1 = CPW1 // S1
CHUNK2, CPW2 = 128, 82
EP = NW * EPW
R = 10112
RPT = R // NS
BR = 1000
EPS = 1e-16


def _tc_a(x_ref, w_ref, as_ref, ad_ref, oh_ref, od_ref, ok_ref):
    i = pl.program_id(0)
    h = jnp.dot(x_ref[...], w_ref[...], preferred_element_type=jnp.float32)
    asrc = jnp.dot(h, as_ref[...], preferred_element_type=jnp.float32)
    adst = jnp.dot(h, ad_ref[...], preferred_element_type=jnp.float32)
    z8 = jnp.zeros((h.shape[0], 8), jnp.float32)
    oh_ref[...] = jnp.concatenate([h, asrc, z8], axis=1)
    od_ref[...] = jnp.concatenate([adst, z8], axis=1)
    cur = jnp.concatenate(
        [jnp.max(asrc, axis=0, keepdims=True),
         jnp.max(adst, axis=0, keepdims=True)], axis=1)

    @pl.when(i == 0)
    def _():
        ok_ref[...] = cur

    @pl.when(i != 0)
    def _():
        ok_ref[...] = jnp.maximum(ok_ref[...], cur)


def _tc_b(p0_ref, p1_ref, b1_ref, rb_ref, w2_ref, a2_ref, oh_ref, oa_ref, ok_ref):
    i = pl.program_id(0)
    acc = p0_ref[...] + p1_ref[...]
    dexp = jnp.dot(acc[:, 128:144], rb_ref[...], preferred_element_type=jnp.float32)
    o1 = acc[:, :128] / (dexp + EPS) + b1_ref[...]
    e = jnp.where(o1 > 0, o1, jnp.exp(o1) - 1.0)
    h2 = jnp.dot(e, w2_ref[...], preferred_element_type=jnp.float32)
    a2 = jnp.dot(h2, a2_ref[...], preferred_element_type=jnp.float32)
    oh_ref[...] = h2
    oa_ref[...] = a2
    cur = jnp.max(a2, axis=0, keepdims=True)

    @pl.when(i == 0)
    def _():
        ok_ref[...] = cur

    @pl.when(i != 0)
    def _():
        ok_ref[...] = jnp.maximum(ok_ref[...], cur)


def _tc_c(p0_ref, p1_ref, b2_ref, rc_ref, o_ref):
    acc = p0_ref[...] + p1_ref[...]
    dexp = jnp.dot(acc[:, 64:80], rc_ref[...], preferred_element_type=jnp.float32)
    o_ref[...] = acc[:, :64] / (dexp + EPS) + b2_ref[...]


_MESH = plsc.VectorSubcoreMesh(core_axis_name="c", subcore_axis_name="s")


@functools.partial(
    pl.kernel,
    out_type=jax.ShapeDtypeStruct((NC, R, D1), jnp.float32),
    mesh=_MESH,
    scratch_types=[
        pltpu.VMEM((2, S1, CHUNK1), jnp.int32),
        pltpu.VMEM((2, S1, CHUNK1), jnp.int32),
        pltpu.VMEM((2, CHUNK1, D1), jnp.float32),
        pltpu.VMEM((2, CHUNK1, 16), jnp.float32),
        pltpu.VMEM((2, CHUNK1, D1), jnp.float32),
        pltpu.VMEM((16,), jnp.float32),
        pltpu.VMEM_SHARED((R, D1), jnp.float32),
        pltpu.SemaphoreType.DMA,
        pltpu.SemaphoreType.DMA,
        pltpu.SemaphoreType.DMA,
        pltpu.SemaphoreType.DMA,
    ],
    compiler_params=pltpu.CompilerParams(use_tc_tiling_on_sc=False),
)
def _sc_layer1(hext, adstt, srcp, dstp, kvec, out,
               idxs, idxd, rows, arows, stage, kv, acc, gsa, gsb, ssem, isem):
    cid = lax.axis_index("c")
    sid = lax.axis_index("s")
    wid = cid * NS + sid
    row0 = sid * RPT
    zv = jnp.zeros((16,), jnp.float32)
    def zrow(e, c2):
        for k in range(D1 // 16):
            stage[0, e, pl.ds(k * 16, 16)] = zv
            stage[1, e, pl.ds(k * 16, 16)] = zv
        return c2
    lax.fori_loop(0, CHUNK1, zrow, 0)
    for q in range(9):
        pltpu.sync_copy(stage.at[0], acc.at[pl.ds(row0 + q * CHUNK1, CHUNK1)])
    pltpu.sync_copy(stage.at[0, pl.ds(0, RPT - 9 * CHUNK1)],
                    acc.at[pl.ds(row0 + 9 * CHUNK1, RPT - 9 * CHUNK1)])
    pltpu.sync_copy(kvec, kv)
    plsc.subcore_barrier()
    kvv = kv[...]
    gsem = (gsa, gsb)

    def issue_idx_async(sup, ib):
        pltpu.async_copy(srcp.at[wid, pl.ds(sup * S1, S1)], idxs.at[ib], isem)
        pltpu.async_copy(dstp.at[wid, pl.ds(sup * S1, S1)], idxd.at[ib], isem)

    def wait_idx(ib):
        pltpu.make_async_copy(srcp.at[wid, pl.ds(0, S1)], idxs.at[ib], isem).wait()
        pltpu.make_async_copy(dstp.at[wid, pl.ds(0, S1)], idxd.at[ib], isem).wait()

    def issue_gather(ib, j, b):
        pltpu.async_copy(hext.at[idxs.at[ib, j]], rows.at[b], gsem[b])
        pltpu.async_copy(adstt.at[idxd.at[ib, j]], arows.at[b], gsem[b])

    def compute(b):
        def edge(e, c2):
            asrc = rows[b, e, pl.ds(128, 16)]
            s = asrc + arows[b, e, :]
            s = jnp.where(s >= 0, s, 0.2 * s)
            alpha = jnp.exp(s - kvv)
            stage[b, e, pl.ds(128, 16)] = alpha
            for hh in range(H1):
                ab = jnp.broadcast_to(alpha[hh], (16,))
                stage[b, e, pl.ds(hh * 16, 16)] = \
                    rows[b, e, pl.ds(hh * 16, 16)] * ab
            return c2

        lax.fori_loop(0, CHUNK1, edge, 0)

    pltpu.sync_copy(srcp.at[wid, pl.ds(0, S1)], idxs.at[0])
    pltpu.sync_copy(dstp.at[wid, pl.ds(0, S1)], idxd.at[0])
    issue_gather(0, 0, 0)

    def body(s, carry):
        B = lax.rem(s, 2)
        for j in range(S1):
            b = j % 2
            ci = s * S1 + j
            pltpu.make_async_copy(hext.at[idxs.at[B, j]], rows.at[b], gsem[b]).wait()
            pltpu.make_async_copy(adstt.at[idxd.at[B, j]], arows.at[b], gsem[b]).wait()

            @pl.when(ci >= 1)
            def _():
                pltpu.make_async_copy(
                    stage.at[1 - b], acc.at[idxd.at[B, 0]], ssem).wait()

            if j == 0:
                @pl.when(s + 1 < NSUP1)
                def _():
                    issue_idx_async(s + 1, 1 - B)

            if j < S1 - 1:
                issue_gather(B, j + 1, 1 - b)
            else:
                @pl.when(ci + 1 < CPW1)
                def _():
                    wait_idx(1 - B)
                    issue_gather(1 - B, 0, 1 - b)

            compute(b)
            pltpu.async_copy(stage.at[b], acc.at[idxd.at[B, j]], ssem, add=True)
        return carry

    lax.fori_loop(0, NSUP1, body, 0)
    pltpu.make_async_copy(stage.at[1], acc.at[idxd.at[0, 0]], ssem).wait()
    plsc.subcore_barrier()
    pltpu.sync_copy(acc.at[pl.ds(row0, RPT)], out.at[cid, pl.ds(row0, RPT)])


@functools.partial(
    pl.kernel,
    out_type=jax.ShapeDtypeStruct((NC, R, D2), jnp.float32),
    mesh=_MESH,
    scratch_types=[
        pltpu.VMEM((CPW2, CHUNK2), jnp.int32),
        pltpu.VMEM((CPW2, CHUNK2), jnp.int32),
        pltpu.VMEM((2, CHUNK2, F2), jnp.float32),
        pltpu.VMEM((2, CHUNK2, D2), jnp.float32),
        pltpu.VMEM((R,), jnp.float32),
        pltpu.VMEM((R,), jnp.float32),
        pltpu.VMEM((16,), jnp.float32),
        pltpu.VMEM((16,), jnp.float32),
        pltpu.VMEM_SHARED((R, D2), jnp.float32),
        pltpu.SemaphoreType.DMA,
        pltpu.SemaphoreType.DMA,
        pltpu.SemaphoreType.DMA,
    ],
    compiler_params=pltpu.CompilerParams(
        use_tc_tiling_on_sc=False, needs_layout_passes=False),
)
def _sc_layer2(h2tab, a2s, a2d, srcp, dstp, kvec, oneh, out,
               idxs, idxd, rows, stage, a2sv, a2dv, kv, ohv, acc, gsa, gsb, ssem):
    cid = lax.axis_index("c")
    sid = lax.axis_index("s")
    wid = cid * NS + sid
    row0 = sid * RPT
    zv = jnp.zeros((16,), jnp.float32)
    def zrow(e, c2):
        for k in range(D2 // 16):
            stage[0, e, pl.ds(k * 16, 16)] = zv
        return c2
    lax.fori_loop(0, CHUNK2, zrow, 0)
    for q in range(4):
        pltpu.sync_copy(stage.at[0], acc.at[pl.ds(row0 + q * CHUNK2, CHUNK2)])
    pltpu.sync_copy(stage.at[0, pl.ds(0, RPT - 4 * CHUNK2)],
                    acc.at[pl.ds(row0 + 4 * CHUNK2, RPT - 4 * CHUNK2)])
    pltpu.sync_copy(a2s, a2sv)
    pltpu.sync_copy(a2d, a2dv)
    pltpu.sync_copy(kvec, kv)
    pltpu.sync_copy(oneh, ohv)
    pltpu.sync_copy(srcp.at[wid], idxs)
    pltpu.sync_copy(dstp.at[wid], idxd)
    plsc.subcore_barrier()
    kvv = kv[...]
    onehot0 = ohv[...]
    gsem = (gsa, gsb)

    def issue(ci, b):
        pltpu.async_copy(h2tab.at[idxs.at[ci]], rows.at[b], gsem[b])

    def compute(ci, b):
        for g in range(8):
            siv = idxs[ci, pl.ds(g * 16, 16)]
            div = idxd[ci, pl.ds(g * 16, 16)]
            s = plsc.load_gather(a2sv, [siv]) + plsc.load_gather(a2dv, [div])
            s = jnp.where(s >= 0, s, 0.2 * s)
            alpha = jnp.exp(s - kvv)
            for j in range(16):
                e = g * 16 + j
                ab = jnp.broadcast_to(alpha[j], (16,))
                stage[b, e, pl.ds(64, 16)] = ab * onehot0
                for k in range(4):
                    stage[b, e, pl.ds(k * 16, 16)] = \
                        rows[b, e, pl.ds(k * 16, 16)] * ab

    issue(0, 0)

    def body(ci0, carry):
        for b in range(2):
            ci = ci0 * 2 + b
            pltpu.make_async_copy(h2tab.at[idxs.at[ci]], rows.at[b], gsem[b]).wait()

            @pl.when(ci >= 1)
            def _():
                pltpu.make_async_copy(
                    stage.at[1 - b], acc.at[idxd.at[ci]], ssem).wait()

            @pl.when(ci + 1 < CPW2)
            def _():
                issue(ci + 1, 1 - b)

            compute(ci, b)
            pltpu.async_copy(stage.at[b], acc.at[idxd.at[ci]], ssem, add=True)
        return carry

    lax.fori_loop(0, CPW2 // 2, body, 0)
    pltpu.make_async_copy(stage.at[1], acc.at[idxd.at[0]], ssem).wait()
    plsc.subcore_barrier()
    pltpu.sync_copy(acc.at[pl.ds(row0, RPT)], out.at[cid, pl.ds(row0, RPT)])


def kernel(x, edge_index, W1, att_src1, att_dst1, b1, W2, att_src2, att_dst2, b2):
    f32 = jnp.float32
    loop = jnp.arange(N, dtype=jnp.int32)
    src = jnp.concatenate([edge_index[0].astype(jnp.int32), loop])
    dst = jnp.concatenate([edge_index[1].astype(jnp.int32), loop])
    padn = EP - E_TOT
    srcp = jnp.concatenate([src, jnp.zeros((padn,), jnp.int32)])
    trash = N + (jnp.arange(padn, dtype=jnp.int32) % (R - N))
    dstp = jnp.concatenate([dst, trash])
    srcp1 = srcp.reshape(NW, CPW1, CHUNK1)
    dstp1 = dstp.reshape(NW, CPW1, CHUNK1)
    srcp2 = srcp.reshape(NW, CPW2, CHUNK2)
    dstp2 = dstp.reshape(NW, CPW2, CHUNK2)

    a1s = att_src1.reshape(H1, C1)
    a1d = att_dst1.reshape(H1, C1)
    eye8 = jnp.eye(H1, dtype=f32)
    As1 = (eye8[:, None, :] * a1s[:, :, None]).reshape(FIN, H1)
    Ad1 = (eye8[:, None, :] * a1d[:, :, None]).reshape(FIN, H1)
    Rb = jnp.concatenate([jnp.repeat(eye8, C1, axis=1), jnp.zeros((8, 128), f32)], axis=0)
    A2 = jnp.zeros((F2, 16), f32)
    A2 = A2.at[:, 0].set(att_src2.reshape(F2))
    A2 = A2.at[:, 1].set(att_dst2.reshape(F2))
    Rc = jnp.concatenate([jnp.ones((1, F2), f32), jnp.zeros((15, F2), f32)], axis=0)

    grid = N // BR
    h1ext, adst_t, k1acc = pl.pallas_call(
        _tc_a,
        grid=(grid,),
        in_specs=[
            pl.BlockSpec((BR, FIN), lambda i: (i, 0)),
            pl.BlockSpec((FIN, FIN), lambda i: (0, 0)),
            pl.BlockSpec((FIN, H1), lambda i: (0, 0)),
            pl.BlockSpec((FIN, H1), lambda i: (0, 0)),
        ],
        out_specs=[
            pl.BlockSpec((BR, D1), lambda i: (i, 0)),
            pl.BlockSpec((BR, 16), lambda i: (i, 0)),
            pl.BlockSpec((1, 16), lambda i: (0, 0)),
        ],
        out_shape=[
            jax.ShapeDtypeStruct((N, D1), f32),
            jax.ShapeDtypeStruct((N, 16), f32),
            jax.ShapeDtypeStruct((1, 16), f32),
        ],
    )(x, W1, As1, Ad1)

    adst_tab = jnp.pad(adst_t, ((0, R - N), (0, 0)))
    k1 = k1acc[0, 0:8] + k1acc[0, 8:16]
    k1vec = jnp.concatenate([k1, jnp.zeros((8,), f32)])

    parts1 = _sc_layer1(h1ext, adst_tab, srcp1, dstp1, k1vec)

    h2tab, a2t, k2acc = pl.pallas_call(
        _tc_b,
        grid=(grid,),
        in_specs=[
            pl.BlockSpec((BR, D1), lambda i: (i, 0)),
            pl.BlockSpec((BR, D1), lambda i: (i, 0)),
            pl.BlockSpec((1, 128), lambda i: (0, 0)),
            pl.BlockSpec((16, 128), lambda i: (0, 0)),
            pl.BlockSpec((128, F2), lambda i: (0, 0)),
            pl.BlockSpec((F2, 16), lambda i: (0, 0)),
        ],
        out_specs=[
            pl.BlockSpec((BR, F2), lambda i: (i, 0)),
            pl.BlockSpec((BR, 16), lambda i: (i, 0)),
            pl.BlockSpec((1, 16), lambda i: (0, 0)),
        ],
        out_shape=[
            jax.ShapeDtypeStruct((N, F2), f32),
            jax.ShapeDtypeStruct((N, 16), f32),
            jax.ShapeDtypeStruct((1, 16), f32),
        ],
    )(parts1[0], parts1[1], b1.reshape(1, 128), Rb, W2, A2)

    a2src = jnp.pad(a2t[:, 0], (0, R - N))
    a2dst = jnp.pad(a2t[:, 1], (0, R - N))
    k2vec = jnp.full((16,), k2acc[0, 0] + k2acc[0, 1], f32)
    onehot = jnp.zeros((16,), f32).at[0].set(1.0)

    parts2 = _sc_layer2(h2tab, a2src, a2dst, srcp2, dstp2, k2vec, onehot)

    out = pl.pallas_call(
        _tc_c,
        grid=(grid,),
        in_specs=[
            pl.BlockSpec((BR, D2), lambda i: (i, 0)),
            pl.BlockSpec((BR, D2), lambda i: (i, 0)),
            pl.BlockSpec((1, F2), lambda i: (0, 0)),
            pl.BlockSpec((16, F2), lambda i: (0, 0)),
        ],
        out_specs=pl.BlockSpec((BR, F2), lambda i: (i, 0)),
        out_shape=jax.ShapeDtypeStruct((N, F2), f32),
    )(parts2[0], parts2[1], b2.reshape(1, F2), Rc)
    return out


# speedup vs baseline: 1.0201x; 1.0201x over previous
import functools

import jax
import jax.numpy as jnp
from jax import lax
from jax.experimental import pallas as pl
from jax.experimental.pallas import tpu as pltpu
from jax.experimental.pallas import tpu_sc as plsc

N = 10000
FIN = 128
H1, C1 = 8, 16
D1 = 144
F2 = 64
D2 = 80
E_RAW = 320000
E_TOT = E_RAW + N
NC, NS, NW = 2, 16, 32
EPW = 10496
CHUNK1, CPW1 = 64, 164
S1 = 4
NSUP1 = CPW1 // S1
CHUNK2, CPW2 = 128, 82
EP = NW * EPW
R = 10112
RPT = R // NS
BR = 1000
EPS = 1e-16


def _tc_a(x_ref, w_ref, as_ref, ad_ref, oh_ref, od_ref, ok_ref):
    i = pl.program_id(0)
    h = jnp.dot(x_ref[...], w_ref[...], preferred_element_type=jnp.float32)
    asrc = jnp.dot(h, as_ref[...], preferred_element_type=jnp.float32)
    adst = jnp.dot(h, ad_ref[...], preferred_element_type=jnp.float32)
    z8 = jnp.zeros((h.shape[0], 8), jnp.float32)
    z24 = jnp.zeros((h.shape[0], 24), jnp.float32)
    oh_ref[...] = jnp.concatenate([h, asrc, z24], axis=1).astype(jnp.bfloat16)
    od_ref[...] = jnp.concatenate([adst, z8], axis=1)
    cur = jnp.concatenate(
        [jnp.max(asrc, axis=0, keepdims=True),
         jnp.max(adst, axis=0, keepdims=True)], axis=1)

    @pl.when(i == 0)
    def _():
        ok_ref[...] = cur

    @pl.when(i != 0)
    def _():
        ok_ref[...] = jnp.maximum(ok_ref[...], cur)


def _tc_b(p0_ref, p1_ref, b1_ref, rb_ref, w2_ref, a2_ref, oh_ref, oa_ref, ok_ref):
    i = pl.program_id(0)
    acc = p0_ref[...] + p1_ref[...]
    dexp = jnp.dot(acc[:, 128:144], rb_ref[...], preferred_element_type=jnp.float32)
    o1 = acc[:, :128] / (dexp + EPS) + b1_ref[...]
    e = jnp.where(o1 > 0, o1, jnp.exp(o1) - 1.0)
    h2 = jnp.dot(e, w2_ref[...], preferred_element_type=jnp.float32)
    a2 = jnp.dot(h2, a2_ref[...], preferred_element_type=jnp.float32)
    oh_ref[...] = h2.astype(jnp.bfloat16)
    oa_ref[...] = a2
    cur = jnp.max(a2, axis=0, keepdims=True)

    @pl.when(i == 0)
    def _():
        ok_ref[...] = cur

    @pl.when(i != 0)
    def _():
        ok_ref[...] = jnp.maximum(ok_ref[...], cur)


def _tc_c(p0_ref, p1_ref, b2_ref, rc_ref, o_ref):
    acc = p0_ref[...] + p1_ref[...]
    dexp = jnp.dot(acc[:, 64:80], rc_ref[...], preferred_element_type=jnp.float32)
    o_ref[...] = acc[:, :64] / (dexp + EPS) + b2_ref[...]


_MESH = plsc.VectorSubcoreMesh(core_axis_name="c", subcore_axis_name="s")


@functools.partial(
    pl.kernel,
    out_type=jax.ShapeDtypeStruct((NC, R, D1), jnp.float32),
    mesh=_MESH,
    scratch_types=[
        pltpu.VMEM((2, S1, CHUNK1), jnp.int32),
        pltpu.VMEM((2, S1, CHUNK1), jnp.int32),
        pltpu.VMEM((2, CHUNK1, 160), jnp.bfloat16),
        pltpu.VMEM((2, CHUNK1, 16), jnp.float32),
        pltpu.VMEM((2, CHUNK1, D1), jnp.float32),
        pltpu.VMEM((16,), jnp.float32),
        pltpu.VMEM_SHARED((R, D1), jnp.float32),
        pltpu.SemaphoreType.DMA,
        pltpu.SemaphoreType.DMA,
        pltpu.SemaphoreType.DMA,
        pltpu.SemaphoreType.DMA,
    ],
    compiler_params=pltpu.CompilerParams(
        use_tc_tiling_on_sc=False, needs_layout_passes=False),
)
def _sc_layer1(hext, adstt, srcp, dstp, kvec, out,
               idxs, idxd, rows, arows, stage, kv, acc, gsa, gsb, ssem, isem):
    cid = lax.axis_index("c")
    sid = lax.axis_index("s")
    wid = cid * NS + sid
    row0 = sid * RPT
    zv = jnp.zeros((16,), jnp.float32)
    def zrow(e, c2):
        for k in range(D1 // 16):
            stage[0, e, pl.ds(k * 16, 16)] = zv
            stage[1, e, pl.ds(k * 16, 16)] = zv
        return c2
    lax.fori_loop(0, CHUNK1, zrow, 0)
    for q in range(9):
        pltpu.sync_copy(stage.at[0], acc.at[pl.ds(row0 + q * CHUNK1, CHUNK1)])
    pltpu.sync_copy(stage.at[0, pl.ds(0, RPT - 9 * CHUNK1)],
                    acc.at[pl.ds(row0 + 9 * CHUNK1, RPT - 9 * CHUNK1)])
    pltpu.sync_copy(kvec, kv)
    plsc.subcore_barrier()
    kvv = kv[...]
    gsem = (gsa, gsb)

    def issue_idx_async(sup, ib):
        pltpu.async_copy(srcp.at[wid, pl.ds(sup * S1, S1)], idxs.at[ib], isem)
        pltpu.async_copy(dstp.at[wid, pl.ds(sup * S1, S1)], idxd.at[ib], isem)

    def wait_idx(ib):
        pltpu.make_async_copy(srcp.at[wid, pl.ds(0, S1)], idxs.at[ib], isem).wait()
        pltpu.make_async_copy(dstp.at[wid, pl.ds(0, S1)], idxd.at[ib], isem).wait()

    def issue_gather(ib, j, b):
        pltpu.async_copy(hext.at[idxs.at[ib, j]], rows.at[b], gsem[b])
        pltpu.async_copy(adstt.at[idxd.at[ib, j]], arows.at[b], gsem[b])

    def compute(b):
        def edge(e, c2):
            asrc, _ = plsc.unpack(rows[b, e, pl.ds(128, 32)],
                                  format=plsc.PackFormat.INTERLEAVED)
            s = asrc + arows[b, e, :]
            s = jnp.where(s >= 0, s, 0.2 * s)
            alpha = jnp.exp(s - kvv)
            stage[b, e, pl.ds(128, 16)] = alpha
            for k in range(4):
                ha, hb = plsc.unpack(rows[b, e, pl.ds(k * 32, 32)],
                                     format=plsc.PackFormat.INTERLEAVED)
                a0 = jnp.broadcast_to(alpha[2 * k], (16,))
                a1 = jnp.broadcast_to(alpha[2 * k + 1], (16,))
                stage[b, e, pl.ds(k * 32, 16)] = ha * a0
                stage[b, e, pl.ds(k * 32 + 16, 16)] = hb * a1
            return c2

        lax.fori_loop(0, CHUNK1, edge, 0)

    pltpu.sync_copy(srcp.at[wid, pl.ds(0, S1)], idxs.at[0])
    pltpu.sync_copy(dstp.at[wid, pl.ds(0, S1)], idxd.at[0])
    issue_gather(0, 0, 0)

    def body(s, carry):
        B = lax.rem(s, 2)
        for j in range(S1):
            b = j % 2
            ci = s * S1 + j
            pltpu.make_async_copy(hext.at[idxs.at[B, j]], rows.at[b], gsem[b]).wait()
            pltpu.make_async_copy(adstt.at[idxd.at[B, j]], arows.at[b], gsem[b]).wait()

            @pl.when(ci >= 1)
            def _():
                pltpu.make_async_copy(
                    stage.at[1 - b], acc.at[idxd.at[B, 0]], ssem).wait()

            if j == 0:
                @pl.when(s + 1 < NSUP1)
                def _():
                    issue_idx_async(s + 1, 1 - B)

            if j < S1 - 1:
                issue_gather(B, j + 1, 1 - b)
            else:
                @pl.when(ci + 1 < CPW1)
                def _():
                    wait_idx(1 - B)
                    issue_gather(1 - B, 0, 1 - b)

            compute(b)
            pltpu.async_copy(stage.at[b], acc.at[idxd.at[B, j]], ssem, add=True)
        return carry

    lax.fori_loop(0, NSUP1, body, 0)
    pltpu.make_async_copy(stage.at[1], acc.at[idxd.at[0, 0]], ssem).wait()
    plsc.subcore_barrier()
    pltpu.sync_copy(acc.at[pl.ds(row0, RPT)], out.at[cid, pl.ds(row0, RPT)])


@functools.partial(
    pl.kernel,
    out_type=jax.ShapeDtypeStruct((NC, R, D2), jnp.float32),
    mesh=_MESH,
    scratch_types=[
        pltpu.VMEM((CPW2, CHUNK2), jnp.int32),
        pltpu.VMEM((CPW2, CHUNK2), jnp.int32),
        pltpu.VMEM((2, CHUNK2, F2), jnp.bfloat16),
        pltpu.VMEM((2, CHUNK2, D2), jnp.float32),
        pltpu.VMEM((R,), jnp.float32),
        pltpu.VMEM((R,), jnp.float32),
        pltpu.VMEM((16,), jnp.float32),
        pltpu.VMEM((16,), jnp.float32),
        pltpu.VMEM_SHARED((R, D2), jnp.float32),
        pltpu.SemaphoreType.DMA,
        pltpu.SemaphoreType.DMA,
        pltpu.SemaphoreType.DMA,
    ],
    compiler_params=pltpu.CompilerParams(
        use_tc_tiling_on_sc=False, needs_layout_passes=False),
)
def _sc_layer2(h2tab, a2s, a2d, srcp, dstp, kvec, oneh, out,
               idxs, idxd, rows, stage, a2sv, a2dv, kv, ohv, acc, gsa, gsb, ssem):
    cid = lax.axis_index("c")
    sid = lax.axis_index("s")
    wid = cid * NS + sid
    row0 = sid * RPT
    zv = jnp.zeros((16,), jnp.float32)
    def zrow(e, c2):
        for k in range(D2 // 16):
            stage[0, e, pl.ds(k * 16, 16)] = zv
        return c2
    lax.fori_loop(0, CHUNK2, zrow, 0)
    for q in range(4):
        pltpu.sync_copy(stage.at[0], acc.at[pl.ds(row0 + q * CHUNK2, CHUNK2)])
    pltpu.sync_copy(stage.at[0, pl.ds(0, RPT - 4 * CHUNK2)],
                    acc.at[pl.ds(row0 + 4 * CHUNK2, RPT - 4 * CHUNK2)])
    pltpu.sync_copy(a2s, a2sv)
    pltpu.sync_copy(a2d, a2dv)
    pltpu.sync_copy(kvec, kv)
    pltpu.sync_copy(oneh, ohv)
    pltpu.sync_copy(srcp.at[wid], idxs)
    pltpu.sync_copy(dstp.at[wid], idxd)
    plsc.subcore_barrier()
    kvv = kv[...]
    onehot0 = ohv[...]
    gsem = (gsa, gsb)

    def issue(ci, b):
        pltpu.async_copy(h2tab.at[idxs.at[ci]], rows.at[b], gsem[b])

    def compute(ci, b):
        for g in range(8):
            siv = idxs[ci, pl.ds(g * 16, 16)]
            div = idxd[ci, pl.ds(g * 16, 16)]
            s = plsc.load_gather(a2sv, [siv]) + plsc.load_gather(a2dv, [div])
            s = jnp.where(s >= 0, s, 0.2 * s)
            alpha = jnp.exp(s - kvv)
            for j in range(16):
                e = g * 16 + j
                ab = jnp.broadcast_to(alpha[j], (16,))
                stage[b, e, pl.ds(64, 16)] = ab * onehot0
                for k in range(2):
                    ha, hb = plsc.unpack(rows[b, e, pl.ds(k * 32, 32)],
                                         format=plsc.PackFormat.INTERLEAVED)
                    stage[b, e, pl.ds(k * 32, 16)] = ha * ab
                    stage[b, e, pl.ds(k * 32 + 16, 16)] = hb * ab

    issue(0, 0)

    def body(ci0, carry):
        for b in range(2):
            ci = ci0 * 2 + b
            pltpu.make_async_copy(h2tab.at[idxs.at[ci]], rows.at[b], gsem[b]).wait()

            @pl.when(ci >= 1)
            def _():
                pltpu.make_async_copy(
                    stage.at[1 - b], acc.at[idxd.at[ci]], ssem).wait()

            @pl.when(ci + 1 < CPW2)
            def _():
                issue(ci + 1, 1 - b)

            compute(ci, b)
            pltpu.async_copy(stage.at[b], acc.at[idxd.at[ci]], ssem, add=True)
        return carry

    lax.fori_loop(0, CPW2 // 2, body, 0)
    pltpu.make_async_copy(stage.at[1], acc.at[idxd.at[0]], ssem).wait()
    plsc.subcore_barrier()
    pltpu.sync_copy(acc.at[pl.ds(row0, RPT)], out.at[cid, pl.ds(row0, RPT)])


def kernel(x, edge_index, W1, att_src1, att_dst1, b1, W2, att_src2, att_dst2, b2):
    f32 = jnp.float32
    loop = jnp.arange(N, dtype=jnp.int32)
    src = jnp.concatenate([edge_index[0].astype(jnp.int32), loop])
    dst = jnp.concatenate([edge_index[1].astype(jnp.int32), loop])
    padn = EP - E_TOT
    srcp = jnp.concatenate([src, jnp.zeros((padn,), jnp.int32)])
    trash = N + (jnp.arange(padn, dtype=jnp.int32) % (R - N))
    dstp = jnp.concatenate([dst, trash])
    srcp1 = srcp.reshape(NW, CPW1, CHUNK1)
    dstp1 = dstp.reshape(NW, CPW1, CHUNK1)
    srcp2 = srcp.reshape(NW, CPW2, CHUNK2)
    dstp2 = dstp.reshape(NW, CPW2, CHUNK2)

    a1s = att_src1.reshape(H1, C1)
    a1d = att_dst1.reshape(H1, C1)
    eye8 = jnp.eye(H1, dtype=f32)
    As1 = (eye8[:, None, :] * a1s[:, :, None]).reshape(FIN, H1)
    Ad1 = (eye8[:, None, :] * a1d[:, :, None]).reshape(FIN, H1)
    Rb = jnp.concatenate([jnp.repeat(eye8, C1, axis=1), jnp.zeros((8, 128), f32)], axis=0)
    A2 = jnp.zeros((F2, 16), f32)
    A2 = A2.at[:, 0].set(att_src2.reshape(F2))
    A2 = A2.at[:, 1].set(att_dst2.reshape(F2))
    Rc = jnp.concatenate([jnp.ones((1, F2), f32), jnp.zeros((15, F2), f32)], axis=0)

    grid = N // BR
    h1ext, adst_t, k1acc = pl.pallas_call(
        _tc_a,
        grid=(grid,),
        in_specs=[
            pl.BlockSpec((BR, FIN), lambda i: (i, 0)),
            pl.BlockSpec((FIN, FIN), lambda i: (0, 0)),
            pl.BlockSpec((FIN, H1), lambda i: (0, 0)),
            pl.BlockSpec((FIN, H1), lambda i: (0, 0)),
        ],
        out_specs=[
            pl.BlockSpec((BR, 160), lambda i: (i, 0)),
            pl.BlockSpec((BR, 16), lambda i: (i, 0)),
            pl.BlockSpec((1, 16), lambda i: (0, 0)),
        ],
        out_shape=[
            jax.ShapeDtypeStruct((N, 160), jnp.bfloat16),
            jax.ShapeDtypeStruct((N, 16), f32),
            jax.ShapeDtypeStruct((1, 16), f32),
        ],
    )(x, W1, As1, Ad1)
    h1ext = h1ext.reshape(N, 5, 2, 16).transpose(0, 1, 3, 2).reshape(N, 160)

    adst_tab = jnp.pad(adst_t, ((0, R - N), (0, 0)))
    k1 = k1acc[0, 0:8] + k1acc[0, 8:16]
    k1vec = jnp.concatenate([k1, jnp.zeros((8,), f32)])

    parts1 = _sc_layer1(h1ext, adst_tab, srcp1, dstp1, k1vec)

    h2tab, a2t, k2acc = pl.pallas_call(
        _tc_b,
        grid=(grid,),
        in_specs=[
            pl.BlockSpec((BR, D1), lambda i: (i, 0)),
            pl.BlockSpec((BR, D1), lambda i: (i, 0)),
            pl.BlockSpec((1, 128), lambda i: (0, 0)),
            pl.BlockSpec((16, 128), lambda i: (0, 0)),
            pl.BlockSpec((128, F2), lambda i: (0, 0)),
            pl.BlockSpec((F2, 16), lambda i: (0, 0)),
        ],
        out_specs=[
            pl.BlockSpec((BR, F2), lambda i: (i, 0)),
            pl.BlockSpec((BR, 16), lambda i: (i, 0)),
            pl.BlockSpec((1, 16), lambda i: (0, 0)),
        ],
        out_shape=[
            jax.ShapeDtypeStruct((N, F2), jnp.bfloat16),
            jax.ShapeDtypeStruct((N, 16), f32),
            jax.ShapeDtypeStruct((1, 16), f32),
        ],
    )(parts1[0], parts1[1], b1.reshape(1, 128), Rb, W2, A2)
    h2tab = h2tab.reshape(N, 2, 2, 16).transpose(0, 1, 3, 2).reshape(N, F2)

    a2src = jnp.pad(a2t[:, 0], (0, R - N))
    a2dst = jnp.pad(a2t[:, 1], (0, R - N))
    k2vec = jnp.full((16,), k2acc[0, 0] + k2acc[0, 1], f32)
    onehot = jnp.zeros((16,), f32).at[0].set(1.0)

    parts2 = _sc_layer2(h2tab, a2src, a2dst, srcp2, dstp2, k2vec, onehot)

    out = pl.pallas_call(
        _tc_c,
        grid=(grid,),
        in_specs=[
            pl.BlockSpec((BR, D2), lambda i: (i, 0)),
            pl.BlockSpec((BR, D2), lambda i: (i, 0)),
            pl.BlockSpec((1, F2), lambda i: (0, 0)),
            pl.BlockSpec((16, F2), lambda i: (0, 0)),
        ],
        out_specs=pl.BlockSpec((BR, F2), lambda i: (i, 0)),
        out_shape=jax.ShapeDtypeStruct((N, F2), f32),
    )(parts2[0], parts2[1], b2.reshape(1, F2), Rc)
    return out


# speedup vs baseline: 1.0886x; 1.0671x over previous
import functools

import jax
import jax.numpy as jnp
from jax import lax
from jax.experimental import pallas as pl
from jax.experimental.pallas import tpu as pltpu
from jax.experimental.pallas import tpu_sc as plsc

N = 10000
FIN = 128
H1, C1 = 8, 16
D1 = 144
F2 = 64
D2 = 80
E_RAW = 320000
E_TOT = E_RAW + N
NC, NS, NW = 2, 16, 32
EPW = 10496
CHUNK1, CPW1 = 64, 164
S1 = 4
NSUP1 = CPW1 // S1
CHUNK2, CPW2 = 128, 82
EP = NW * EPW
R = 10112
RPT = R // NS
BR = 1000
EPS = 1e-16


def _tc_a(x_ref, w_ref, as_ref, ad_ref, oh_ref, od_ref, ok_ref):
    i = pl.program_id(0)
    h = jnp.dot(x_ref[...], w_ref[...], preferred_element_type=jnp.float32)
    asrc = jnp.dot(h, as_ref[...], preferred_element_type=jnp.float32)
    adst = jnp.dot(h, ad_ref[...], preferred_element_type=jnp.float32)
    z8 = jnp.zeros((h.shape[0], 8), jnp.float32)
    oh_ref[...] = jnp.concatenate([h, asrc, z8], axis=1)
    od_ref[...] = jnp.concatenate([adst, z8], axis=1)
    cur = jnp.concatenate(
        [jnp.max(asrc, axis=0, keepdims=True),
         jnp.max(adst, axis=0, keepdims=True)], axis=1)

    @pl.when(i == 0)
    def _():
        ok_ref[...] = cur

    @pl.when(i != 0)
    def _():
        ok_ref[...] = jnp.maximum(ok_ref[...], cur)


def _tc_b(p0_ref, p1_ref, b1_ref, rb_ref, w2_ref, a2_ref, oh_ref, oa_ref, ok_ref):
    i = pl.program_id(0)
    acc = p0_ref[...] + p1_ref[...]
    dexp = jnp.dot(acc[:, 128:144], rb_ref[...], preferred_element_type=jnp.float32)
    o1 = acc[:, :128] / (dexp + EPS) + b1_ref[...]
    e = jnp.where(o1 > 0, o1, jnp.exp(o1) - 1.0)
    h2 = jnp.dot(e, w2_ref[...], preferred_element_type=jnp.float32)
    a2 = jnp.dot(h2, a2_ref[...], preferred_element_type=jnp.float32)
    oh_ref[...] = h2.astype(jnp.bfloat16)
    oa_ref[...] = a2
    cur = jnp.max(a2, axis=0, keepdims=True)

    @pl.when(i == 0)
    def _():
        ok_ref[...] = cur

    @pl.when(i != 0)
    def _():
        ok_ref[...] = jnp.maximum(ok_ref[...], cur)


def _tc_c(p0_ref, p1_ref, b2_ref, rc_ref, o_ref):
    acc = p0_ref[...] + p1_ref[...]
    dexp = jnp.dot(acc[:, 64:80], rc_ref[...], preferred_element_type=jnp.float32)
    o_ref[...] = acc[:, :64] / (dexp + EPS) + b2_ref[...]


_MESH = plsc.VectorSubcoreMesh(core_axis_name="c", subcore_axis_name="s")


@functools.partial(
    pl.kernel,
    out_type=jax.ShapeDtypeStruct((NC, R, D1), jnp.float32),
    mesh=_MESH,
    scratch_types=[
        pltpu.VMEM((2, S1, CHUNK1), jnp.int32),
        pltpu.VMEM((2, S1, CHUNK1), jnp.int32),
        pltpu.VMEM((2, CHUNK1, D1), jnp.float32),
        pltpu.VMEM((2, CHUNK1, 16), jnp.float32),
        pltpu.VMEM((2, CHUNK1, D1), jnp.float32),
        pltpu.VMEM((16,), jnp.float32),
        pltpu.VMEM_SHARED((R, D1), jnp.float32),
        pltpu.SemaphoreType.DMA,
        pltpu.SemaphoreType.DMA,
        pltpu.SemaphoreType.DMA,
        pltpu.SemaphoreType.DMA,
    ],
    compiler_params=pltpu.CompilerParams(
        use_tc_tiling_on_sc=False, needs_layout_passes=False),
)
def _sc_layer1(hext, adstt, srcp, dstp, kvec, out,
               idxs, idxd, rows, arows, stage, kv, acc, gsa, gsb, ssem, isem):
    cid = lax.axis_index("c")
    sid = lax.axis_index("s")
    wid = cid * NS + sid
    row0 = sid * RPT
    zv = jnp.zeros((16,), jnp.float32)
    def zrow(e, c2):
        for k in range(D1 // 16):
            stage[0, e, pl.ds(k * 16, 16)] = zv
            stage[1, e, pl.ds(k * 16, 16)] = zv
        return c2
    lax.fori_loop(0, CHUNK1, zrow, 0)
    for q in range(9):
        pltpu.sync_copy(stage.at[0], acc.at[pl.ds(row0 + q * CHUNK1, CHUNK1)])
    pltpu.sync_copy(stage.at[0, pl.ds(0, RPT - 9 * CHUNK1)],
                    acc.at[pl.ds(row0 + 9 * CHUNK1, RPT - 9 * CHUNK1)])
    pltpu.sync_copy(kvec, kv)
    plsc.subcore_barrier()
    kvv = kv[...]
    gsem = (gsa, gsb)

    def issue_idx_async(sup, ib):
        pltpu.async_copy(srcp.at[wid, pl.ds(sup * S1, S1)], idxs.at[ib], isem)
        pltpu.async_copy(dstp.at[wid, pl.ds(sup * S1, S1)], idxd.at[ib], isem)

    def wait_idx(ib):
        pltpu.make_async_copy(srcp.at[wid, pl.ds(0, S1)], idxs.at[ib], isem).wait()
        pltpu.make_async_copy(dstp.at[wid, pl.ds(0, S1)], idxd.at[ib], isem).wait()

    def issue_gather(ib, j, b):
        pltpu.async_copy(hext.at[idxs.at[ib, j]], rows.at[b], gsem[b])
        pltpu.async_copy(adstt.at[idxd.at[ib, j]], arows.at[b], gsem[b])

    def compute(b):
        def edge(e, c2):
            asrc = rows[b, e, pl.ds(128, 16)]
            s = asrc + arows[b, e, :]
            s = jnp.where(s >= 0, s, 0.2 * s)
            alpha = jnp.exp(s - kvv)
            stage[b, e, pl.ds(128, 16)] = alpha
            for hh in range(H1):
                ab = jnp.broadcast_to(alpha[hh], (16,))
                stage[b, e, pl.ds(hh * 16, 16)] = \
                    rows[b, e, pl.ds(hh * 16, 16)] * ab
            return c2

        lax.fori_loop(0, CHUNK1, edge, 0)

    pltpu.sync_copy(srcp.at[wid, pl.ds(0, S1)], idxs.at[0])
    pltpu.sync_copy(dstp.at[wid, pl.ds(0, S1)], idxd.at[0])
    issue_gather(0, 0, 0)

    def body(s, carry):
        B = lax.rem(s, 2)
        for j in range(S1):
            b = j % 2
            ci = s * S1 + j
            pltpu.make_async_copy(hext.at[idxs.at[B, j]], rows.at[b], gsem[b]).wait()
            pltpu.make_async_copy(adstt.at[idxd.at[B, j]], arows.at[b], gsem[b]).wait()

            @pl.when(ci >= 1)
            def _():
                pltpu.make_async_copy(
                    stage.at[1 - b], acc.at[idxd.at[B, 0]], ssem).wait()

            if j == 0:
                @pl.when(s + 1 < NSUP1)
                def _():
                    issue_idx_async(s + 1, 1 - B)

            if j < S1 - 1:
                issue_gather(B, j + 1, 1 - b)
            else:
                @pl.when(ci + 1 < CPW1)
                def _():
                    wait_idx(1 - B)
                    issue_gather(1 - B, 0, 1 - b)

            compute(b)
            pltpu.async_copy(stage.at[b], acc.at[idxd.at[B, j]], ssem, add=True)
        return carry

    lax.fori_loop(0, NSUP1, body, 0)
    pltpu.make_async_copy(stage.at[1], acc.at[idxd.at[0, 0]], ssem).wait()
    plsc.subcore_barrier()
    pltpu.sync_copy(acc.at[pl.ds(row0, RPT)], out.at[cid, pl.ds(row0, RPT)])


@functools.partial(
    pl.kernel,
    out_type=jax.ShapeDtypeStruct((NC, R, D2), jnp.float32),
    mesh=_MESH,
    scratch_types=[
        pltpu.VMEM((CPW2, CHUNK2), jnp.int32),
        pltpu.VMEM((CPW2, CHUNK2), jnp.int32),
        pltpu.VMEM((2, CHUNK2, F2), jnp.bfloat16),
        pltpu.VMEM((2, CHUNK2, D2), jnp.float32),
        pltpu.VMEM((R,), jnp.float32),
        pltpu.VMEM((R,), jnp.float32),
        pltpu.VMEM((16,), jnp.float32),
        pltpu.VMEM((16,), jnp.float32),
        pltpu.VMEM_SHARED((R, D2), jnp.float32),
        pltpu.SemaphoreType.DMA,
        pltpu.SemaphoreType.DMA,
        pltpu.SemaphoreType.DMA,
    ],
    compiler_params=pltpu.CompilerParams(
        use_tc_tiling_on_sc=False, needs_layout_passes=False),
)
def _sc_layer2(h2tab, a2s, a2d, srcp, dstp, kvec, oneh, out,
               idxs, idxd, rows, stage, a2sv, a2dv, kv, ohv, acc, gsa, gsb, ssem):
    cid = lax.axis_index("c")
    sid = lax.axis_index("s")
    wid = cid * NS + sid
    row0 = sid * RPT
    zv = jnp.zeros((16,), jnp.float32)
    def zrow(e, c2):
        for k in range(D2 // 16):
            stage[0, e, pl.ds(k * 16, 16)] = zv
        return c2
    lax.fori_loop(0, CHUNK2, zrow, 0)
    for q in range(4):
        pltpu.sync_copy(stage.at[0], acc.at[pl.ds(row0 + q * CHUNK2, CHUNK2)])
    pltpu.sync_copy(stage.at[0, pl.ds(0, RPT - 4 * CHUNK2)],
                    acc.at[pl.ds(row0 + 4 * CHUNK2, RPT - 4 * CHUNK2)])
    pltpu.sync_copy(a2s, a2sv)
    pltpu.sync_copy(a2d, a2dv)
    pltpu.sync_copy(kvec, kv)
    pltpu.sync_copy(oneh, ohv)
    pltpu.sync_copy(srcp.at[wid], idxs)
    pltpu.sync_copy(dstp.at[wid], idxd)
    plsc.subcore_barrier()
    kvv = kv[...]
    onehot0 = ohv[...]
    gsem = (gsa, gsb)

    def issue(ci, b):
        pltpu.async_copy(h2tab.at[idxs.at[ci]], rows.at[b], gsem[b])

    def compute(ci, b):
        for g in range(8):
            siv = idxs[ci, pl.ds(g * 16, 16)]
            div = idxd[ci, pl.ds(g * 16, 16)]
            s = plsc.load_gather(a2sv, [siv]) + plsc.load_gather(a2dv, [div])
            s = jnp.where(s >= 0, s, 0.2 * s)
            alpha = jnp.exp(s - kvv)
            for j in range(16):
                e = g * 16 + j
                ab = jnp.broadcast_to(alpha[j], (16,))
                stage[b, e, pl.ds(64, 16)] = ab * onehot0
                for k in range(2):
                    ha, hb = plsc.unpack(rows[b, e, pl.ds(k * 32, 32)],
                                         format=plsc.PackFormat.INTERLEAVED)
                    stage[b, e, pl.ds(k * 32, 16)] = ha * ab
                    stage[b, e, pl.ds(k * 32 + 16, 16)] = hb * ab

    issue(0, 0)

    def body(ci0, carry):
        for b in range(2):
            ci = ci0 * 2 + b
            pltpu.make_async_copy(h2tab.at[idxs.at[ci]], rows.at[b], gsem[b]).wait()

            @pl.when(ci >= 1)
            def _():
                pltpu.make_async_copy(
                    stage.at[1 - b], acc.at[idxd.at[ci]], ssem).wait()

            @pl.when(ci + 1 < CPW2)
            def _():
                issue(ci + 1, 1 - b)

            compute(ci, b)
            pltpu.async_copy(stage.at[b], acc.at[idxd.at[ci]], ssem, add=True)
        return carry

    lax.fori_loop(0, CPW2 // 2, body, 0)
    pltpu.make_async_copy(stage.at[1], acc.at[idxd.at[0]], ssem).wait()
    plsc.subcore_barrier()
    pltpu.sync_copy(acc.at[pl.ds(row0, RPT)], out.at[cid, pl.ds(row0, RPT)])


def kernel(x, edge_index, W1, att_src1, att_dst1, b1, W2, att_src2, att_dst2, b2):
    f32 = jnp.float32
    loop = jnp.arange(N, dtype=jnp.int32)
    src = jnp.concatenate([edge_index[0].astype(jnp.int32), loop])
    dst = jnp.concatenate([edge_index[1].astype(jnp.int32), loop])
    padn = EP - E_TOT
    srcp = jnp.concatenate([src, jnp.zeros((padn,), jnp.int32)])
    trash = N + (jnp.arange(padn, dtype=jnp.int32) % (R - N))
    dstp = jnp.concatenate([dst, trash])
    srcp1 = srcp.reshape(NW, CPW1, CHUNK1)
    dstp1 = dstp.reshape(NW, CPW1, CHUNK1)
    srcp2 = srcp.reshape(NW, CPW2, CHUNK2)
    dstp2 = dstp.reshape(NW, CPW2, CHUNK2)

    a1s = att_src1.reshape(H1, C1)
    a1d = att_dst1.reshape(H1, C1)
    eye8 = jnp.eye(H1, dtype=f32)
    As1 = (eye8[:, None, :] * a1s[:, :, None]).reshape(FIN, H1)
    Ad1 = (eye8[:, None, :] * a1d[:, :, None]).reshape(FIN, H1)
    Rb = jnp.concatenate([jnp.repeat(eye8, C1, axis=1), jnp.zeros((8, 128), f32)], axis=0)
    A2 = jnp.zeros((F2, 16), f32)
    A2 = A2.at[:, 0].set(att_src2.reshape(F2))
    A2 = A2.at[:, 1].set(att_dst2.reshape(F2))
    Rc = jnp.concatenate([jnp.ones((1, F2), f32), jnp.zeros((15, F2), f32)], axis=0)

    grid = N // BR
    h1ext, adst_t, k1acc = pl.pallas_call(
        _tc_a,
        grid=(grid,),
        in_specs=[
            pl.BlockSpec((BR, FIN), lambda i: (i, 0)),
            pl.BlockSpec((FIN, FIN), lambda i: (0, 0)),
            pl.BlockSpec((FIN, H1), lambda i: (0, 0)),
            pl.BlockSpec((FIN, H1), lambda i: (0, 0)),
        ],
        out_specs=[
            pl.BlockSpec((BR, D1), lambda i: (i, 0)),
            pl.BlockSpec((BR, 16), lambda i: (i, 0)),
            pl.BlockSpec((1, 16), lambda i: (0, 0)),
        ],
        out_shape=[
            jax.ShapeDtypeStruct((N, D1), f32),
            jax.ShapeDtypeStruct((N, 16), f32),
            jax.ShapeDtypeStruct((1, 16), f32),
        ],
    )(x, W1, As1, Ad1)

    adst_tab = jnp.pad(adst_t, ((0, R - N), (0, 0)))
    k1 = k1acc[0, 0:8] + k1acc[0, 8:16]
    k1vec = jnp.concatenate([k1, jnp.zeros((8,), f32)])

    parts1 = _sc_layer1(h1ext, adst_tab, srcp1, dstp1, k1vec)

    h2tab, a2t, k2acc = pl.pallas_call(
        _tc_b,
        grid=(grid,),
        in_specs=[
            pl.BlockSpec((BR, D1), lambda i: (i, 0)),
            pl.BlockSpec((BR, D1), lambda i: (i, 0)),
            pl.BlockSpec((1, 128), lambda i: (0, 0)),
            pl.BlockSpec((16, 128), lambda i: (0, 0)),
            pl.BlockSpec((128, F2), lambda i: (0, 0)),
            pl.BlockSpec((F2, 16), lambda i: (0, 0)),
        ],
        out_specs=[
            pl.BlockSpec((BR, F2), lambda i: (i, 0)),
            pl.BlockSpec((BR, 16), lambda i: (i, 0)),
            pl.BlockSpec((1, 16), lambda i: (0, 0)),
        ],
        out_shape=[
            jax.ShapeDtypeStruct((N, F2), jnp.bfloat16),
            jax.ShapeDtypeStruct((N, 16), f32),
            jax.ShapeDtypeStruct((1, 16), f32),
        ],
    )(parts1[0], parts1[1], b1.reshape(1, 128), Rb, W2, A2)
    h2tab = h2tab.reshape(N, 2, 2, 16).transpose(0, 1, 3, 2).reshape(N, F2)

    a2src = jnp.pad(a2t[:, 0], (0, R - N))
    a2dst = jnp.pad(a2t[:, 1], (0, R - N))
    k2vec = jnp.full((16,), k2acc[0, 0] + k2acc[0, 1], f32)
    onehot = jnp.zeros((16,), f32).at[0].set(1.0)

    parts2 = _sc_layer2(h2tab, a2src, a2dst, srcp2, dstp2, k2vec, onehot)

    out = pl.pallas_call(
        _tc_c,
        grid=(grid,),
        in_specs=[
            pl.BlockSpec((BR, D2), lambda i: (i, 0)),
            pl.BlockSpec((BR, D2), lambda i: (i, 0)),
            pl.BlockSpec((1, F2), lambda i: (0, 0)),
            pl.BlockSpec((16, F2), lambda i: (0, 0)),
        ],
        out_specs=pl.BlockSpec((BR, F2), lambda i: (i, 0)),
        out_shape=jax.ShapeDtypeStruct((N, F2), f32),
    )(parts2[0], parts2[1], b2.reshape(1, F2), Rc)
    return out
